# baseline (speedup 1.0000x reference)
import functools

import jax
import jax.numpy as jnp
from jax import lax
from jax.experimental import pallas as pl
from jax.experimental.pallas import tpu as pltpu
from jax.experimental.pallas import tpu_sc as plsc

_L = 16
_WORKERS = 32
_LANE = 128
_SUB = 8
_GROUPS = _LANE // _L
_KB = 5


def _make_sc_kernel(B, K, NT):
  KTU = K // _SUB
  NCH = KTU // _KB
  OUT_PER_W = _GROUPS * 3 * _L
  NTR = NT * _L

  mesh = plsc.VectorSubcoreMesh(core_axis_name="c", subcore_axis_name="s")

  @functools.partial(
      pl.kernel,
      out_type=jax.ShapeDtypeStruct((_WORKERS * OUT_PER_W,), jnp.float32),
      mesh=mesh,
      compiler_params=pltpu.CompilerParams(needs_layout_passes=False),
      scratch_types=[
          pltpu.VMEM((_KB, _SUB, _LANE), jnp.int32),
          pltpu.VMEM((_KB, _SUB, _LANE), jnp.int32),
          pltpu.VMEM((_KB, _SUB, _LANE), jnp.int32),
          pltpu.VMEM((_KB, _SUB, _LANE), jnp.int32),
          pltpu.VMEM((_GROUPS * 3 * _L,), jnp.float32),
          pltpu.VMEM((NTR,), jnp.float32),
          pltpu.VMEM((NTR,), jnp.float32),
          pltpu.VMEM((NTR,), jnp.float32),
          pltpu.VMEM((NTR,), jnp.float32),
          pltpu.VMEM((NTR,), jnp.float32),
          pltpu.VMEM((NTR,), jnp.float32),
          pltpu.SemaphoreType.DMA,
          pltpu.SemaphoreType.DMA,
          pltpu.SemaphoreType.DMA,
          pltpu.SemaphoreType.DMA,
      ],
  )
  def sc(xv_ref, u0_ref, u1_ref, u2_ref, j0_ref, j1_ref, j2_ref, out_ref,
         ubuf0, ubuf1, jbuf0, jbuf1, obuf, tu0, tu1, tu2, tj0, tj1, tj2,
         semu0, semu1, semj0, semj1):
    wid = lax.axis_index("s") * 2 + lax.axis_index("c")

    pltpu.sync_copy(u0_ref, tu0)
    pltpu.sync_copy(u1_ref, tu1)
    pltpu.sync_copy(u2_ref, tu2)
    pltpu.sync_copy(j0_ref, tj0)
    pltpu.sync_copy(j1_ref, tj1)
    pltpu.sync_copy(j2_ref, tj2)

    zf = jnp.zeros((_L,), jnp.float32)
    lane = lax.iota(jnp.int32, _L)

    for g in range(_GROUPS):
      ob = g * 3 * _L
      obuf[pl.ds(ob, _L)] = zf
      obuf[pl.ds(ob + _L, _L)] = zf
      obuf[pl.ds(ob + 2 * _L, _L)] = zf

    def start_u(c, buf, sem):
      pltpu.async_copy(xv_ref.at[pl.ds(c * _KB, _KB), wid, :, :], buf, sem)

    def start_j(c, buf, sem):
      pltpu.async_copy(
          xv_ref.at[pl.ds(KTU + c * _KB, _KB), wid, :, :], buf, sem)

    def wait(buf, sem):
      pltpu.make_async_copy(xv_ref.at[pl.ds(0, _KB), wid, :, :],
                            buf, sem).wait()

    def compute(ub, jb):
      for g in range(_GROUPS):
        gcol = g * _L

        def kti_body(t, acc):
          d, us, js = acc
          for ks in range(_SUB):
            ua = lax.shift_left(ub[t, ks, pl.ds(gcol, _L)], 4) + lane
            ja = lax.shift_left(jb[t, ks, pl.ds(gcol, _L)], 4) + lane
            u0 = plsc.load_gather(tu0, [ua])
            u1 = plsc.load_gather(tu1, [ua])
            u2 = plsc.load_gather(tu2, [ua])
            j0 = plsc.load_gather(tj0, [ja])
            j1 = plsc.load_gather(tj1, [ja])
            j2 = plsc.load_gather(tj2, [ja])
            d = d + (u0 * j0 + u1 * j1 + u2 * j2)
            us = us + (u0 * u0 + u1 * u1 + u2 * u2)
            js = js + (j0 * j0 + j1 * j1 + j2 * j2)
          return (d, us, js)

        d, us, js = lax.fori_loop(0, _KB, kti_body, (zf, zf, zf))

        ob = g * 3 * _L
        obuf[pl.ds(ob, _L)] = obuf[pl.ds(ob, _L)] + d
        obuf[pl.ds(ob + _L, _L)] = obuf[pl.ds(ob + _L, _L)] + us
        obuf[pl.ds(ob + 2 * _L, _L)] = obuf[pl.ds(ob + 2 * _L, _L)] + js

    start_u(0, ubuf0, semu0)
    start_j(0, jbuf0, semj0)
    start_u(1, ubuf1, semu1)
    start_j(1, jbuf1, semj1)

    def chunk_pair(i, carry):
      c0 = i * 2
      wait(ubuf0, semu0)
      wait(jbuf0, semj0)
      compute(ubuf0, jbuf0)
      start_u(c0 + 2, ubuf0, semu0)
      start_j(c0 + 2, jbuf0, semj0)
      wait(ubuf1, semu1)
      wait(jbuf1, semj1)
      compute(ubuf1, jbuf1)
      cn = jnp.minimum(c0 + 3, NCH - 1)
      start_u(cn, ubuf1, semu1)
      start_j(cn, jbuf1, semj1)
      return carry

    lax.fori_loop(0, (NCH - 1) // 2, chunk_pair, 0)

    wait(ubuf0, semu0)
    wait(jbuf0, semj0)
    compute(ubuf0, jbuf0)
    wait(ubuf1, semu1)
    wait(jbuf1, semj1)

    pltpu.sync_copy(obuf, out_ref.at[pl.ds(wid * OUT_PER_W, OUT_PER_W)])

  return sc


def _tc_head(p_ref, w_ref, b_ref, o_ref):
  p = p_ref[...]
  d = p[:, 0:_L]
  us = p[:, _L:2 * _L]
  js = p[:, 2 * _L:3 * _L]
  inv = lax.rsqrt(jnp.maximum(us, 1e-12)) * lax.rsqrt(jnp.maximum(js, 1e-12))
  z = d * inv * w_ref[0, 0] + b_ref[0, 0]
  o_ref[...] = jax.nn.sigmoid(z)


def kernel(x, user_table, joke_table, W_out, b_out):
  B = x.shape[0]
  n_users = user_table.shape[0]
  K2 = x.shape[1]
  K = K2 // 2
  NT = 1024

  xv = x.reshape(B // _LANE, _LANE, K2 // _SUB, _SUB).transpose(2, 0, 3, 1)

  ut = jnp.pad(user_table, ((0, NT - n_users), (0, 0)))
  jt = jnp.pad(joke_table, ((0, NT - joke_table.shape[0]), (0, 0)))

  def rep(col):
    return jnp.broadcast_to(col[:, None], (NT, _L)).reshape(NT * _L)

  sc = _make_sc_kernel(B, K, NT)
  p = sc(
      xv,
      rep(ut[:, 0]), rep(ut[:, 1]), rep(ut[:, 2]),
      rep(jt[:, 0]), rep(jt[:, 1]), rep(jt[:, 2]),
  ).reshape(B // _L, 3 * _L)

  out = pl.pallas_call(
      _tc_head,
      out_shape=jax.ShapeDtypeStruct((B // _L, _L), jnp.float32),
  )(p, W_out, b_out.reshape(1, 1))
  return out.reshape(B, 1)
